# Initial kernel scaffold; baseline (speedup 1.0000x reference)
import jax
import jax.numpy as jnp
from jax.experimental import pallas as pl


def kernel(x, w1, b1, w2, b2, wf1, bf1, wf2, bf2):
    raise NotImplementedError("write your pallas kernel here")


# speedup vs baseline: 2.5182x; 2.5182x over previous
import jax
import jax.numpy as jnp
from jax.experimental import pallas as pl
from jax.experimental.pallas import tpu as pltpu

_C2 = 128
_H = 512
_CO = 128


def _fused_kernel(x_ref, w1_ref, b1_ref, w2_ref, b2_ref, wf1_ref, bf1_ref,
                  wf2_ref, bf2_ref, o_ref, p1_ref, feat_ref):
    f32 = jnp.float32
    TB = x_ref.shape[0]

    for ho in range(15):
        base = 2 * ho * 32
        acc = None
        for k in range(9):
            ky, kx = divmod(k, 3)
            xs = x_ref[:, pl.ds(base + ky * 32 + kx, 64), :].reshape(TB * 64, 3)
            t = jnp.dot(xs, w1_ref[k], preferred_element_type=f32)
            acc = t if acc is None else acc + t
        acc = jnp.maximum(acc + b1_ref[...], 0.0)
        a = acc.reshape(TB, 64, 20)
        m = jnp.maximum(a[:, :32, :], a[:, 32:, :])
        m = m.reshape(TB * 16, 2, 20)
        m = jnp.maximum(m[:, 0, :], m[:, 1, :])
        m = m.reshape(TB, 16, 20)[:, :15, :]
        p1_ref[:, pl.ds(ho * 15, 15), :] = m

    for ho in range(5):
        base = 30 * ho
        acc2 = None
        for k in range(25):
            ky, kx = divmod(k, 5)
            ps = p1_ref[:, pl.ds(base + ky * 15 + kx, 30), :].reshape(TB * 30, 20)
            t = jnp.dot(ps, w2_ref[k], preferred_element_type=f32)
            acc2 = t if acc2 is None else acc2 + t
        acc2 = jnp.maximum(acc2 + b2_ref[...], 0.0)
        a2 = acc2.reshape(TB, 30, 128)
        m2 = jnp.maximum(a2[:, :15, :], a2[:, 15:, :])
        m2 = m2[:, :10, :].reshape(TB * 5, 2, 128)
        m2 = jnp.maximum(m2[:, 0, :], m2[:, 1, :])
        feat_ref[:, pl.ds(5 * ho, 5), :] = m2.reshape(TB, 5, 128)

    f = feat_ref[...].reshape(TB, 25 * 128)
    h = jnp.dot(f, wf1_ref[...], preferred_element_type=f32) + bf1_ref[...]
    h = jnp.maximum(h, 0.0)
    o_ref[...] = jnp.dot(h, wf2_ref[...], preferred_element_type=f32) + bf2_ref[...]


def kernel(x, w1, b1, w2, b2, wf1, bf1, wf2, bf2):
    B = x.shape[0]
    xf = jnp.transpose(x, (0, 2, 3, 1)).reshape(B, 1024, 3)
    xf = jnp.pad(xf, ((0, 0), (0, 8), (0, 0)))

    w2p = jnp.pad(w2, ((0, 0), (0, 0), (0, _C2 - w2.shape[2])))
    b2p = jnp.pad(b2, ((0, 0), (0, _C2 - b2.shape[1])))
    D, H = wf1.shape
    C = wf2.shape[1]
    w1p = wf1.reshape(25, 50, H)
    w1p = jnp.pad(w1p, ((0, 0), (0, _C2 - 50), (0, _H - H))).reshape(25 * _C2, _H)
    b1p = jnp.pad(bf1, ((0, 0), (0, _H - H)))
    wf2p = jnp.pad(wf2, ((0, _H - H), (0, _CO - C)))
    b2fp = jnp.pad(bf2, ((0, 0), (0, _CO - C)))

    TB = 16
    while B % TB:
        TB //= 2
    grid = B // TB

    flops = 2 * B * (960 * 27 * 20 + 150 * 500 * _C2 + 25 * _C2 * _H + _H * _CO)
    bytes_accessed = 4 * (B * (1032 * 3 + _CO) + w1.size + b1.size
                          + w2p.size + b2p.size + w1p.size + b1p.size
                          + wf2p.size + b2fp.size)

    out = pl.pallas_call(
        _fused_kernel,
        out_shape=jax.ShapeDtypeStruct((B, _CO), jnp.float32),
        grid=(grid,),
        in_specs=[
            pl.BlockSpec((TB, 1032, 3), lambda i: (i, 0, 0)),
            pl.BlockSpec((9, 3, 20), lambda i: (0, 0, 0)),
            pl.BlockSpec((1, 20), lambda i: (0, 0)),
            pl.BlockSpec((25, 20, _C2), lambda i: (0, 0, 0)),
            pl.BlockSpec((1, _C2), lambda i: (0, 0)),
            pl.BlockSpec((25 * _C2, _H), lambda i: (0, 0)),
            pl.BlockSpec((1, _H), lambda i: (0, 0)),
            pl.BlockSpec((_H, _CO), lambda i: (0, 0)),
            pl.BlockSpec((1, _CO), lambda i: (0, 0)),
        ],
        out_specs=pl.BlockSpec((TB, _CO), lambda i: (i, 0)),
        scratch_shapes=[
            pltpu.VMEM((TB, 225, 20), jnp.float32),
            pltpu.VMEM((TB, 25, _C2), jnp.float32),
        ],
        compiler_params=pltpu.CompilerParams(
            dimension_semantics=("parallel",),
            vmem_limit_bytes=100 * 1024 * 1024),
        cost_estimate=pl.CostEstimate(flops=flops, transcendentals=0,
                                      bytes_accessed=bytes_accessed),
    )(xf, w1, b1, w2p, b2p, w1p, b1p, wf2p, b2fp)
    return out[:, :C]


# speedup vs baseline: 27.4258x; 10.8909x over previous
import jax
import jax.numpy as jnp
from jax.experimental import pallas as pl
from jax.experimental.pallas import tpu as pltpu

_TB = 128


def _toeplitz(vfull, J, S):
    L = vfull.shape[0]
    stack = jnp.pad(vfull, ((0, S + 1 - L),) + ((0, 0),) * (vfull.ndim - 1))
    reps = -(-(J * S) // (S + 1)) + 1
    flat = jnp.tile(stack, (reps,) + (1,) * (vfull.ndim - 1))
    return flat[:J * S].reshape((J, S) + vfull.shape[1:])


def _fused_kernel(x_ref, w1t_ref, b1t_ref, w2t_ref, b2t_ref, wf1t_ref,
                  bf1t_ref, wf2t_ref, bf2t_ref, o_ref, p1_ref, feat_ref):
    f32 = jnp.float32

    for ho in range(15):
        xs = x_ref[pl.ds(192 * ho, 392), :]
        out = jnp.dot(w1t_ref[...], xs, preferred_element_type=f32)
        out = jnp.maximum(out + b1t_ref[...], 0.0)
        m = jnp.maximum(out[:640, :], out[640:, :])
        m = m.reshape(16, 2, 20, _TB)
        m = jnp.maximum(m[:, 0], m[:, 1])
        m = m[:15].reshape(300, _TB)
        p1_ref[pl.ds(300 * ho, 300), :] = m

    for h2 in range(5):
        ps = p1_ref[pl.ds(600 * h2, 1880), :]
        out = jnp.dot(w2t_ref[...], ps, preferred_element_type=f32)
        out = jnp.maximum(out + b2t_ref[...], 0.0)
        m = jnp.maximum(out[:750, :], out[750:, :])
        m = m[:500].reshape(5, 2, 50, _TB)
        m = jnp.maximum(m[:, 0], m[:, 1])
        feat_ref[pl.ds(250 * h2, 250), :] = m.reshape(250, _TB)

    f = feat_ref[...]
    h = jnp.dot(wf1t_ref[...], f, preferred_element_type=f32) + bf1t_ref[...]
    h = jnp.maximum(h, 0.0)
    o_ref[...] = jnp.dot(wf2t_ref[...], h,
                         preferred_element_type=f32) + bf2t_ref[...]


def kernel(x, w1, b1, w2, b2, wf1, bf1, wf2, bf2):
    B = x.shape[0]
    C = wf2.shape[1]
    Bp = -(-B // _TB) * _TB

    x2 = jnp.transpose(x, (2, 3, 1, 0)).reshape(1024, 3, B)
    x2 = jnp.pad(x2, ((0, 8), (0, 0), (0, Bp - B))).reshape(3096, Bp)

    vf1 = jnp.pad(w1.reshape(3, 3, 3, 20), ((0, 0), (0, 29), (0, 0), (0, 0)))
    vf1 = vf1.reshape(96, 3, 20)[:67]
    t1 = _toeplitz(vf1, 64, 130)
    w1t = jnp.pad(t1.transpose(0, 3, 1, 2).reshape(1280, 390),
                  ((0, 0), (0, 2)))
    b1t = jnp.broadcast_to(jnp.tile(b1[0], 64)[:, None], (1280, _TB))

    vf2 = jnp.pad(w2.reshape(5, 5, 20, 50), ((0, 0), (0, 10), (0, 0), (0, 0)))
    vf2 = vf2.reshape(75, 20, 50)[:65]
    t2 = _toeplitz(vf2, 30, 94)
    w2t = t2.transpose(0, 3, 1, 2).reshape(1500, 1880)
    b2t = jnp.broadcast_to(jnp.tile(b2[0], 30)[:, None], (1500, _TB))

    wf1t = jnp.pad(wf1.T, ((0, 512 - wf1.shape[1]), (0, 0)))
    bf1t = jnp.pad(jnp.broadcast_to(bf1[0][:, None], (wf1.shape[1], _TB)),
                   ((0, 512 - wf1.shape[1]), (0, 0)))
    wf2t = jnp.pad(wf2.T, ((0, 16 - C), (0, 512 - wf2.shape[0])))
    bf2t = jnp.pad(jnp.broadcast_to(bf2[0][:, None], (C, _TB)), ((0, 16 - C), (0, 0)))

    grid = Bp // _TB
    flops = 2 * grid * _TB * (15 * 1280 * 392 + 5 * 1500 * 1880
                              + 512 * 1250 + 16 * 512) // _TB * _TB
    bytes_accessed = 4 * (Bp * 3096 + Bp * 16 + w1t.size + w2t.size
                          + wf1t.size + b1t.size + b2t.size)

    out = pl.pallas_call(
        _fused_kernel,
        out_shape=jax.ShapeDtypeStruct((16, Bp), jnp.float32),
        grid=(grid,),
        in_specs=[
            pl.BlockSpec((3096, _TB), lambda i: (0, i)),
            pl.BlockSpec((1280, 392), lambda i: (0, 0)),
            pl.BlockSpec((1280, _TB), lambda i: (0, 0)),
            pl.BlockSpec((1500, 1880), lambda i: (0, 0)),
            pl.BlockSpec((1500, _TB), lambda i: (0, 0)),
            pl.BlockSpec((512, 1250), lambda i: (0, 0)),
            pl.BlockSpec((512, _TB), lambda i: (0, 0)),
            pl.BlockSpec((16, 512), lambda i: (0, 0)),
            pl.BlockSpec((16, _TB), lambda i: (0, 0)),
        ],
        out_specs=pl.BlockSpec((16, _TB), lambda i: (0, i)),
        scratch_shapes=[
            pltpu.VMEM((4500, _TB), jnp.float32),
            pltpu.VMEM((1250, _TB), jnp.float32),
        ],
        compiler_params=pltpu.CompilerParams(
            dimension_semantics=("parallel",),
            vmem_limit_bytes=100 * 1024 * 1024),
        cost_estimate=pl.CostEstimate(flops=flops, transcendentals=0,
                                      bytes_accessed=bytes_accessed),
    )(x2, w1t, b1t, w2t, b2t, wf1t, bf1t, wf2t, bf2t)
    return out[:C, :B].T


# speedup vs baseline: 29.7229x; 1.0838x over previous
import jax
import jax.numpy as jnp
from jax.experimental import pallas as pl
from jax.experimental.pallas import tpu as pltpu

_TB = 128


def _toeplitz(vfull, J, S):
    L = vfull.shape[0]
    stack = jnp.pad(vfull, ((0, S + 1 - L),) + ((0, 0),) * (vfull.ndim - 1))
    reps = -(-(J * S) // (S + 1)) + 1
    flat = jnp.tile(stack, (reps,) + (1,) * (vfull.ndim - 1))
    return flat[:J * S].reshape((J, S) + vfull.shape[1:])


def _fused_kernel(x_ref, w1t_ref, b1t_ref, w2t_ref, b2t_ref, wf1t_ref,
                  bf1t_ref, wf2t_ref, bf2t_ref, o_ref, p1_ref, feat_ref):
    f32 = jnp.float32
    bf16 = jnp.bfloat16

    for ho in range(15):
        xs = x_ref[pl.ds(192 * ho, 392), :]
        out = jnp.dot(w1t_ref[...], xs, preferred_element_type=f32)
        out = jnp.maximum(out + b1t_ref[...], 0.0)
        m = jnp.maximum(out[:640, :], out[640:, :])
        m = m.reshape(16, 2, 20, _TB)
        m = jnp.maximum(m[:, 0], m[:, 1])
        m = m[:15].reshape(300, _TB)
        p1_ref[pl.ds(300 * ho, 300), :] = m

    for h2 in range(5):
        ps = p1_ref[pl.ds(600 * h2, 1880), :].astype(bf16)
        out = jnp.dot(w2t_ref[...], ps, preferred_element_type=f32)
        out = jnp.maximum(out + b2t_ref[...], 0.0)
        m = jnp.maximum(out[:750, :], out[750:, :])
        m = m[:500].reshape(5, 2, 50, _TB)
        m = jnp.maximum(m[:, 0], m[:, 1])
        feat_ref[pl.ds(250 * h2, 250), :] = m.reshape(250, _TB)

    f = feat_ref[...].astype(bf16)
    h = jnp.dot(wf1t_ref[...], f, preferred_element_type=f32) + bf1t_ref[...]
    h = jnp.maximum(h, 0.0).astype(bf16)
    o_ref[...] = jnp.dot(wf2t_ref[...], h,
                         preferred_element_type=f32) + bf2t_ref[...]


def kernel(x, w1, b1, w2, b2, wf1, bf1, wf2, bf2):
    B = x.shape[0]
    C = wf2.shape[1]
    Bp = -(-B // _TB) * _TB

    x2 = jnp.transpose(x.astype(jnp.bfloat16), (2, 3, 1, 0)).reshape(1024, 3, B)
    x2 = jnp.pad(x2, ((0, 8), (0, 0), (0, Bp - B))).reshape(3096, Bp)

    vf1 = jnp.pad(w1.reshape(3, 3, 3, 20), ((0, 0), (0, 29), (0, 0), (0, 0)))
    vf1 = vf1.reshape(96, 3, 20)[:67]
    t1 = _toeplitz(vf1, 64, 130)
    w1t = jnp.pad(t1.transpose(0, 3, 1, 2).reshape(1280, 390),
                  ((0, 0), (0, 2))).astype(jnp.bfloat16)
    b1t = jnp.broadcast_to(jnp.tile(b1[0], 64)[:, None], (1280, _TB))

    vf2 = jnp.pad(w2.reshape(5, 5, 20, 50), ((0, 0), (0, 10), (0, 0), (0, 0)))
    vf2 = vf2.reshape(75, 20, 50)[:65]
    t2 = _toeplitz(vf2, 30, 94)
    w2t = t2.transpose(0, 3, 1, 2).reshape(1500, 1880).astype(jnp.bfloat16)
    b2t = jnp.broadcast_to(jnp.tile(b2[0], 30)[:, None], (1500, _TB))

    wf1t = jnp.pad(wf1.T, ((0, 512 - wf1.shape[1]), (0, 0))).astype(jnp.bfloat16)
    bf1t = jnp.pad(jnp.broadcast_to(bf1[0][:, None], (wf1.shape[1], _TB)),
                   ((0, 512 - wf1.shape[1]), (0, 0)))
    wf2t = jnp.pad(wf2.T, ((0, 16 - C), (0, 512 - wf2.shape[0]))).astype(jnp.bfloat16)
    bf2t = jnp.pad(jnp.broadcast_to(bf2[0][:, None], (C, _TB)), ((0, 16 - C), (0, 0)))

    grid = Bp // _TB
    flops = 2 * grid * _TB * (15 * 1280 * 392 + 5 * 1500 * 1880
                              + 512 * 1250 + 16 * 512) // _TB * _TB
    bytes_accessed = 4 * (Bp * 3096 + Bp * 16 + w1t.size + w2t.size
                          + wf1t.size + b1t.size + b2t.size)

    out = pl.pallas_call(
        _fused_kernel,
        out_shape=jax.ShapeDtypeStruct((16, Bp), jnp.float32),
        grid=(grid,),
        in_specs=[
            pl.BlockSpec((3096, _TB), lambda i: (0, i)),
            pl.BlockSpec((1280, 392), lambda i: (0, 0)),
            pl.BlockSpec((1280, _TB), lambda i: (0, 0)),
            pl.BlockSpec((1500, 1880), lambda i: (0, 0)),
            pl.BlockSpec((1500, _TB), lambda i: (0, 0)),
            pl.BlockSpec((512, 1250), lambda i: (0, 0)),
            pl.BlockSpec((512, _TB), lambda i: (0, 0)),
            pl.BlockSpec((16, 512), lambda i: (0, 0)),
            pl.BlockSpec((16, _TB), lambda i: (0, 0)),
        ],
        out_specs=pl.BlockSpec((16, _TB), lambda i: (0, i)),
        scratch_shapes=[
            pltpu.VMEM((4500, _TB), jnp.float32),
            pltpu.VMEM((1250, _TB), jnp.float32),
        ],
        compiler_params=pltpu.CompilerParams(
            dimension_semantics=("parallel",),
            vmem_limit_bytes=100 * 1024 * 1024),
        cost_estimate=pl.CostEstimate(flops=flops, transcendentals=0,
                                      bytes_accessed=bytes_accessed),
    )(x2, w1t, b1t, w2t, b2t, wf1t, bf1t, wf2t, bf2t)
    return out[:C, :B].T


# speedup vs baseline: 33.9967x; 1.1438x over previous
import jax
import jax.numpy as jnp
from jax.experimental import pallas as pl
from jax.experimental.pallas import tpu as pltpu

_TB = 128


def _toeplitz(vfull, J, S):
    L = vfull.shape[0]
    stack = jnp.pad(vfull, ((0, S + 1 - L),) + ((0, 0),) * (vfull.ndim - 1))
    reps = -(-(J * S) // (S + 1)) + 1
    flat = jnp.tile(stack, (reps,) + (1,) * (vfull.ndim - 1))
    return flat[:J * S].reshape((J, S) + vfull.shape[1:])


def _fused_kernel(x_ref, w1t_ref, b1t_ref, w2t_ref, b2t_ref, wf1t_ref,
                  bf1t_ref, wf2t_ref, bf2t_ref, o_ref, p1_ref, feat_ref):
    f32 = jnp.float32
    bf16 = jnp.bfloat16

    for ho in range(15):
        xs = jnp.concatenate(
            [x_ref[pl.ds(1040 * ci + 64 * ho, 144), :] for ci in range(3)],
            axis=0)
        out = jnp.dot(w1t_ref[...], xs, preferred_element_type=f32)
        out = jnp.maximum(out + b1t_ref[...], 0.0)
        m = jnp.maximum(out[:640, :], out[640:, :])
        m = m.reshape(16, 2, 20, _TB)
        m = jnp.maximum(m[:, 0], m[:, 1])
        m = m[:15].reshape(300, _TB)
        p1_ref[pl.ds(300 * ho, 300), :] = m

    for h2 in range(5):
        ps = p1_ref[pl.ds(600 * h2, 1880), :].astype(bf16)
        out = jnp.dot(w2t_ref[...], ps, preferred_element_type=f32)
        out = jnp.maximum(out + b2t_ref[...], 0.0)
        m = jnp.maximum(out[:750, :], out[750:, :])
        m = m[:500].reshape(5, 2, 50, _TB)
        m = jnp.maximum(m[:, 0], m[:, 1])
        feat_ref[pl.ds(250 * h2, 250), :] = m.reshape(250, _TB)

    f = feat_ref[...].astype(bf16)
    h = jnp.dot(wf1t_ref[...], f, preferred_element_type=f32) + bf1t_ref[...]
    h = jnp.maximum(h, 0.0).astype(bf16)
    o_ref[...] = jnp.dot(wf2t_ref[...], h,
                         preferred_element_type=f32) + bf2t_ref[...]


def kernel(x, w1, b1, w2, b2, wf1, bf1, wf2, bf2):
    B = x.shape[0]
    C = wf2.shape[1]
    Bp = -(-B // _TB) * _TB

    x2 = jnp.pad(x.astype(jnp.bfloat16).reshape(B, 3, 1024),
                 ((0, 0), (0, 0), (0, 16))).reshape(B, 3120).T
    if Bp != B:
        x2 = jnp.pad(x2, ((0, 0), (0, Bp - B)))

    vf1 = jnp.pad(w1.reshape(3, 3, 3, 20), ((0, 0), (0, 29), (0, 0), (0, 0)))
    vf1 = vf1.reshape(96, 3, 20)[:67]
    t1 = _toeplitz(vf1, 64, 130)
    w1t = jnp.pad(t1.transpose(0, 3, 2, 1), ((0, 0), (0, 0), (0, 0), (0, 14)))
    w1t = w1t.reshape(1280, 432).astype(jnp.bfloat16)
    b1t = jnp.broadcast_to(jnp.tile(b1[0], 64)[:, None], (1280, _TB))

    vf2 = jnp.pad(w2.reshape(5, 5, 20, 50), ((0, 0), (0, 10), (0, 0), (0, 0)))
    vf2 = vf2.reshape(75, 20, 50)[:65]
    t2 = _toeplitz(vf2, 30, 94)
    w2t = t2.transpose(0, 3, 1, 2).reshape(1500, 1880).astype(jnp.bfloat16)
    b2t = jnp.broadcast_to(jnp.tile(b2[0], 30)[:, None], (1500, _TB))

    wf1t = jnp.pad(wf1.T, ((0, 512 - wf1.shape[1]), (0, 0))).astype(jnp.bfloat16)
    bf1t = jnp.pad(jnp.broadcast_to(bf1[0][:, None], (wf1.shape[1], _TB)),
                   ((0, 512 - wf1.shape[1]), (0, 0)))
    wf2t = jnp.pad(wf2.T, ((0, 16 - C), (0, 512 - wf2.shape[0]))).astype(jnp.bfloat16)
    bf2t = jnp.pad(jnp.broadcast_to(bf2[0][:, None], (C, _TB)), ((0, 16 - C), (0, 0)))

    grid = Bp // _TB
    flops = 2 * grid * _TB * (15 * 1280 * 392 + 5 * 1500 * 1880
                              + 512 * 1250 + 16 * 512) // _TB * _TB
    bytes_accessed = 4 * (Bp * 3120 + Bp * 16 + w1t.size + w2t.size
                          + wf1t.size + b1t.size + b2t.size)

    out = pl.pallas_call(
        _fused_kernel,
        out_shape=jax.ShapeDtypeStruct((16, Bp), jnp.float32),
        grid=(grid,),
        in_specs=[
            pl.BlockSpec((3120, _TB), lambda i: (0, i)),
            pl.BlockSpec((1280, 432), lambda i: (0, 0)),
            pl.BlockSpec((1280, _TB), lambda i: (0, 0)),
            pl.BlockSpec((1500, 1880), lambda i: (0, 0)),
            pl.BlockSpec((1500, _TB), lambda i: (0, 0)),
            pl.BlockSpec((512, 1250), lambda i: (0, 0)),
            pl.BlockSpec((512, _TB), lambda i: (0, 0)),
            pl.BlockSpec((16, 512), lambda i: (0, 0)),
            pl.BlockSpec((16, _TB), lambda i: (0, 0)),
        ],
        out_specs=pl.BlockSpec((16, _TB), lambda i: (0, i)),
        scratch_shapes=[
            pltpu.VMEM((4500, _TB), jnp.float32),
            pltpu.VMEM((1250, _TB), jnp.float32),
        ],
        compiler_params=pltpu.CompilerParams(
            dimension_semantics=("parallel",),
            vmem_limit_bytes=100 * 1024 * 1024),
        cost_estimate=pl.CostEstimate(flops=flops, transcendentals=0,
                                      bytes_accessed=bytes_accessed),
    )(x2, w1t, b1t, w2t, b2t, wf1t, bf1t, wf2t, bf2t)
    return out[:C, :B].T


# speedup vs baseline: 47.5286x; 1.3980x over previous
import jax
import jax.numpy as jnp
from jax.experimental import pallas as pl
from jax.experimental.pallas import tpu as pltpu

_TB = 256


def _toeplitz(vfull, J, S):
    L = vfull.shape[0]
    stack = jnp.pad(vfull, ((0, S + 1 - L),) + ((0, 0),) * (vfull.ndim - 1))
    reps = -(-(J * S) // (S + 1)) + 1
    flat = jnp.tile(stack, (reps,) + (1,) * (vfull.ndim - 1))
    return flat[:J * S].reshape((J, S) + vfull.shape[1:])


def _fused_kernel(x_ref, w1t_ref, b1t_ref, w2t_ref, b2t_ref, wf1t_ref,
                  bf1t_ref, wf2t_ref, bf2t_ref, o_ref, p1_ref, feat_ref):
    f32 = jnp.float32
    bf16 = jnp.bfloat16

    for ho in range(15):
        xs = jnp.concatenate(
            [x_ref[pl.ds(1040 * ci + 64 * ho, 144), :] for ci in range(3)],
            axis=0)
        out = jnp.dot(w1t_ref[...], xs, preferred_element_type=f32)
        out = jnp.maximum(out + b1t_ref[...], 0.0)
        m = jnp.maximum(out[:640, :], out[640:, :])
        m = m.reshape(16, 2, 20, _TB)
        m = jnp.maximum(m[:, 0], m[:, 1])
        m = m[:15].reshape(300, _TB)
        p1_ref[pl.ds(300 * ho, 300), :] = m

    for h2 in range(5):
        ps = p1_ref[pl.ds(600 * h2, 1880), :].astype(bf16)
        out = jnp.dot(w2t_ref[...], ps, preferred_element_type=f32)
        out = jnp.maximum(out + b2t_ref[...], 0.0)
        m = jnp.maximum(out[:750, :], out[750:, :])
        m = m[:500].reshape(5, 2, 50, _TB)
        m = jnp.maximum(m[:, 0], m[:, 1])
        feat_ref[pl.ds(250 * h2, 250), :] = m.reshape(250, _TB)

    f = feat_ref[...].astype(bf16)
    h = jnp.dot(wf1t_ref[...], f, preferred_element_type=f32) + bf1t_ref[...]
    h = jnp.maximum(h, 0.0).astype(bf16)
    o_ref[...] = jnp.dot(wf2t_ref[...], h,
                         preferred_element_type=f32) + bf2t_ref[...]


def kernel(x, w1, b1, w2, b2, wf1, bf1, wf2, bf2):
    B = x.shape[0]
    C = wf2.shape[1]
    Bp = -(-B // _TB) * _TB

    x2 = jnp.pad(x.astype(jnp.bfloat16).reshape(B, 3, 1024),
                 ((0, 0), (0, 0), (0, 16))).reshape(B, 3120).T
    if Bp != B:
        x2 = jnp.pad(x2, ((0, 0), (0, Bp - B)))

    vf1 = jnp.pad(w1.astype(jnp.bfloat16).reshape(3, 3, 3, 20), ((0, 0), (0, 29), (0, 0), (0, 0)))
    vf1 = vf1.reshape(96, 3, 20)[:67]
    t1 = _toeplitz(vf1, 64, 130)
    w1t = jnp.pad(t1.transpose(0, 3, 2, 1), ((0, 0), (0, 0), (0, 0), (0, 14)))
    w1t = w1t.reshape(1280, 432)
    b1t = jnp.broadcast_to(jnp.tile(b1[0].astype(jnp.bfloat16), 64)[:, None], (1280, _TB))

    vf2 = jnp.pad(w2.astype(jnp.bfloat16).reshape(5, 5, 20, 50), ((0, 0), (0, 10), (0, 0), (0, 0)))
    vf2 = vf2.reshape(75, 20, 50)[:65]
    t2 = _toeplitz(vf2, 30, 94)
    w2t = t2.transpose(0, 3, 1, 2).reshape(1500, 1880)
    b2t = jnp.broadcast_to(jnp.tile(b2[0].astype(jnp.bfloat16), 30)[:, None], (1500, _TB))

    wf1t = jnp.pad(wf1.astype(jnp.bfloat16).T, ((0, 512 - wf1.shape[1]), (0, 0)))
    bf1t = jnp.pad(jnp.broadcast_to(bf1[0].astype(jnp.bfloat16)[:, None], (wf1.shape[1], _TB)),
                   ((0, 512 - wf1.shape[1]), (0, 0)))
    wf2t = jnp.pad(wf2.astype(jnp.bfloat16).T, ((0, 16 - C), (0, 512 - wf2.shape[0])))
    bf2t = jnp.pad(jnp.broadcast_to(bf2[0].astype(jnp.bfloat16)[:, None], (C, _TB)), ((0, 16 - C), (0, 0)))

    grid = Bp // _TB
    flops = 2 * grid * _TB * (15 * 1280 * 392 + 5 * 1500 * 1880
                              + 512 * 1250 + 16 * 512) // _TB * _TB
    bytes_accessed = 4 * (Bp * 3120 + Bp * 16 + w1t.size + w2t.size
                          + wf1t.size + b1t.size + b2t.size)

    out = pl.pallas_call(
        _fused_kernel,
        out_shape=jax.ShapeDtypeStruct((16, Bp), jnp.float32),
        grid=(grid,),
        in_specs=[
            pl.BlockSpec((3120, _TB), lambda i: (0, i)),
            pl.BlockSpec((1280, 432), lambda i: (0, 0)),
            pl.BlockSpec((1280, _TB), lambda i: (0, 0)),
            pl.BlockSpec((1500, 1880), lambda i: (0, 0)),
            pl.BlockSpec((1500, _TB), lambda i: (0, 0)),
            pl.BlockSpec((512, 1250), lambda i: (0, 0)),
            pl.BlockSpec((512, _TB), lambda i: (0, 0)),
            pl.BlockSpec((16, 512), lambda i: (0, 0)),
            pl.BlockSpec((16, _TB), lambda i: (0, 0)),
        ],
        out_specs=pl.BlockSpec((16, _TB), lambda i: (0, i)),
        scratch_shapes=[
            pltpu.VMEM((4500, _TB), jnp.float32),
            pltpu.VMEM((1250, _TB), jnp.float32),
        ],
        compiler_params=pltpu.CompilerParams(
            dimension_semantics=("parallel",),
            vmem_limit_bytes=100 * 1024 * 1024),
        cost_estimate=pl.CostEstimate(flops=flops, transcendentals=0,
                                      bytes_accessed=bytes_accessed),
    )(x2, w1t, b1t, w2t, b2t, wf1t, bf1t, wf2t, bf2t)
    return out[:C, :B].T
